# Initial kernel scaffold; baseline (speedup 1.0000x reference)
import jax
import jax.numpy as jnp
from jax.experimental import pallas as pl


def kernel(indices, values, training):
    raise NotImplementedError("write your pallas kernel here")


# speedup vs baseline: 1.1152x; 1.1152x over previous
import functools

import jax
import jax.numpy as jnp
import numpy as np
from jax import lax
from jax.experimental import pallas as pl
from jax.experimental.pallas import tpu as pltpu
from jax.experimental.pallas import tpu_sc as plsc

_NNZ = 2684354
_NW = 32
_CHUNK = 83904
_LAST = _NNZ - (_NW - 1) * _CHUNK
_NCHUNK16 = _CHUNK // 16

_KS0 = np.uint32(0)
_KS1 = np.uint32(42)
_KS2 = np.uint32(int(_KS0) ^ int(_KS1) ^ 0x1BD11BDA)
_KS = (_KS0, _KS1, _KS2)
_ROT = ((13, 15, 26, 6), (17, 29, 16, 24))


def _rotl(x, r):
    return (x << jnp.uint32(r)) | (x >> jnp.uint32(32 - r))


def _threefry_y0(x1):
    x0 = jnp.full((16,), _KS0, jnp.uint32)
    x1 = x1 + _KS1
    for g in range(5):
        for r in _ROT[g % 2]:
            x0 = x0 + x1
            x1 = _rotl(x1, r)
            x1 = x0 ^ x1
        x0 = x0 + _KS[(g + 1) % 3]
        if g < 4:
            x1 = x1 + (_KS[(g + 2) % 3] + np.uint32(g + 1))
    return x0


_mesh = plsc.VectorSubcoreMesh(core_axis_name="c", subcore_axis_name="s")


@functools.partial(
    pl.kernel,
    out_type=jax.ShapeDtypeStruct((_NNZ,), jnp.float32),
    mesh=_mesh,
    scratch_types=[
        pltpu.VMEM((_CHUNK,), jnp.float32),
        pltpu.VMEM((32,), jnp.float32),
    ],
)
def _sc_dropout(values_hbm, scales_hbm, out_hbm, vbuf, svbuf):
    wid = lax.axis_index("s") * jnp.int32(2) + lax.axis_index("c")
    base = wid * jnp.int32(_CHUNK)
    is_last = wid == jnp.int32(_NW - 1)

    pltpu.sync_copy(scales_hbm, svbuf)
    keep = svbuf[pl.ds(0, 16)]
    drop = svbuf[pl.ds(16, 16)]

    @pl.when(jnp.logical_not(is_last))
    def _():
        pltpu.sync_copy(values_hbm.at[pl.ds(base, _CHUNK)],
                        vbuf.at[pl.ds(0, _CHUNK)])

    @pl.when(is_last)
    def _():
        pltpu.sync_copy(values_hbm.at[pl.ds(base, _LAST)],
                        vbuf.at[pl.ds(0, _LAST)])

    lane = lax.iota(jnp.int32, 16)

    def body(i, carry):
        off = i * jnp.int32(16)
        v = vbuf[pl.ds(off, 16)]
        cnt = lax.convert_element_type(base + off + lane, jnp.uint32)
        y0 = _threefry_y0(cnt)
        m = y0 < jnp.uint32(0x80000000)
        vbuf[pl.ds(off, 16)] = v * jnp.where(m, keep, drop)
        return carry

    lax.fori_loop(jnp.int32(0), jnp.int32(_NCHUNK16), body, jnp.int32(0))

    @pl.when(jnp.logical_not(is_last))
    def _():
        pltpu.sync_copy(vbuf.at[pl.ds(0, _CHUNK)],
                        out_hbm.at[pl.ds(base, _CHUNK)])

    @pl.when(is_last)
    def _():
        pltpu.sync_copy(vbuf.at[pl.ds(0, _LAST)],
                        out_hbm.at[pl.ds(base, _LAST)])


def kernel(indices, values, training):
    t = jnp.asarray(training)
    keep = jnp.where(t != 0, jnp.float32(2.0), jnp.float32(1.0))
    drop = jnp.where(t != 0, jnp.float32(0.0), jnp.float32(1.0))
    scales = jnp.concatenate([
        jnp.broadcast_to(keep.astype(jnp.float32), (16,)),
        jnp.broadcast_to(drop.astype(jnp.float32), (16,)),
    ])
    out_val = _sc_dropout(values, scales)
    return (indices, out_val)
